# baseline (speedup 1.0000x reference)
import functools

import jax
import jax.numpy as jnp
from jax import lax
from jax.experimental import pallas as pl
from jax.experimental.pallas import tpu as pltpu
from jax.experimental.pallas import tpu_sc as plsc

EMBEDDING_DIM = 64
IDX_PER_GATHER = 128
GATHERS_PER_CHUNK = 4
CHUNK = IDX_PER_GATHER * GATHERS_PER_CHUNK


def _build(num_ids: int):
    info = plsc.get_sparse_core_info()
    nc, ns = info.num_cores, info.num_subcores
    nw = nc * ns
    per_w = num_ids // nw
    assert per_w * nw == num_ids and per_w % CHUNK == 0
    nchunks = per_w // CHUNK
    mesh = plsc.VectorSubcoreMesh(core_axis_name="c", subcore_axis_name="s")

    @functools.partial(
        pl.kernel,
        mesh=mesh,
        out_type=jax.ShapeDtypeStruct((num_ids, EMBEDDING_DIM), jnp.float32),
        scratch_types=[
            pltpu.VMEM((CHUNK,), jnp.int32),
            pltpu.VMEM((CHUNK, EMBEDDING_DIM), jnp.float32),
            pltpu.SemaphoreType.DMA,
        ],
        compiler_params=pltpu.CompilerParams(use_tc_tiling_on_sc=False),
    )
    def emb_kernel(ids_hbm, w_hbm, out_hbm, idx_v, rows_v, sem):
        wid = lax.axis_index("s") * nc + lax.axis_index("c")
        base = wid * per_w

        def body(c, carry):
            off = base + c * CHUNK
            pltpu.sync_copy(ids_hbm.at[pl.ds(off, CHUNK)], idx_v)
            copies = []
            for j in range(GATHERS_PER_CHUNK):
                src = w_hbm.at[idx_v.at[pl.ds(j * IDX_PER_GATHER, IDX_PER_GATHER)]]
                dst = rows_v.at[pl.ds(j * IDX_PER_GATHER, IDX_PER_GATHER)]
                copies.append(pltpu.async_copy(src, dst, sem))
            for cp in copies:
                cp.wait()
            pltpu.sync_copy(rows_v, out_hbm.at[pl.ds(off, CHUNK)])
            return carry

        lax.fori_loop(0, nchunks, body, 0)

    return emb_kernel


def kernel(input_ids, weight):
    batch, hist = input_ids.shape
    ids_flat = input_ids.reshape(-1).astype(jnp.int32)
    out = _build(batch * hist)(ids_flat, weight)
    return out.reshape(batch, hist, EMBEDDING_DIM)


# speedup vs baseline: 1.0409x; 1.0409x over previous
import functools

import jax
import jax.numpy as jnp
from jax import lax
from jax.experimental import pallas as pl
from jax.experimental.pallas import tpu as pltpu
from jax.experimental.pallas import tpu_sc as plsc

EMBEDDING_DIM = 64
IDX_PER_GATHER = 128
GATHERS_PER_CHUNK = 4
CHUNK = IDX_PER_GATHER * GATHERS_PER_CHUNK


def _build(num_ids: int):
    info = plsc.get_sparse_core_info()
    nc, ns = info.num_cores, info.num_subcores
    nw = nc * ns
    per_w = num_ids // nw
    assert per_w * nw == num_ids and per_w % CHUNK == 0
    nchunks = per_w // CHUNK
    mesh = plsc.VectorSubcoreMesh(core_axis_name="c", subcore_axis_name="s")

    @functools.partial(
        pl.kernel,
        mesh=mesh,
        out_type=jax.ShapeDtypeStruct((num_ids, EMBEDDING_DIM), jnp.float32),
        scratch_types=[
            pltpu.VMEM((2, CHUNK), jnp.int32),
            pltpu.VMEM((2, CHUNK, EMBEDDING_DIM), jnp.float32),
            pltpu.SemaphoreType.DMA((2,)),
            pltpu.SemaphoreType.DMA((2,)),
            pltpu.SemaphoreType.DMA((2,)),
        ],
        compiler_params=pltpu.CompilerParams(use_tc_tiling_on_sc=False),
    )
    def emb_kernel(ids_hbm, w_hbm, out_hbm, idx_v, rows_v, sem_i, sem_g, sem_o):
        wid = lax.axis_index("s") * nc + lax.axis_index("c")
        base = wid * per_w
        G = IDX_PER_GATHER

        def wait_idx(b):
            pltpu.make_async_copy(
                ids_hbm.at[pl.ds(0, CHUNK)], idx_v.at[b], sem_i.at[b]
            ).wait()

        def wait_gathers(b):
            for j in range(GATHERS_PER_CHUNK):
                pltpu.make_async_copy(
                    out_hbm.at[pl.ds(0, G)],
                    rows_v.at[b, pl.ds(j * G, G)],
                    sem_g.at[b],
                ).wait()

        def wait_out(b):
            pltpu.make_async_copy(
                rows_v.at[b], out_hbm.at[pl.ds(0, CHUNK)], sem_o.at[b]
            ).wait()

        def fire_gathers(b, c):
            for j in range(GATHERS_PER_CHUNK):
                pltpu.async_copy(
                    w_hbm.at[idx_v.at[b, pl.ds(j * G, G)]],
                    rows_v.at[b, pl.ds(j * G, G)],
                    sem_g.at[b],
                )

        pltpu.async_copy(ids_hbm.at[pl.ds(base, CHUNK)], idx_v.at[0], sem_i.at[0])
        pltpu.async_copy(
            ids_hbm.at[pl.ds(base + CHUNK, CHUNK)], idx_v.at[1], sem_i.at[1]
        )

        def step(c, b):
            @pl.when(jnp.logical_and(c >= 1, c <= nchunks))
            def _():
                wait_gathers(1 - b)
                pltpu.async_copy(
                    rows_v.at[1 - b],
                    out_hbm.at[pl.ds(base + (c - 1) * CHUNK, CHUNK)],
                    sem_o.at[1 - b],
                )

                @pl.when(c + 1 < nchunks)
                def _():
                    pltpu.async_copy(
                        ids_hbm.at[pl.ds(base + (c + 1) * CHUNK, CHUNK)],
                        idx_v.at[1 - b],
                        sem_i.at[1 - b],
                    )

            @pl.when(c < nchunks)
            def _():
                @pl.when(c >= 2)
                def _():
                    wait_out(b)

                wait_idx(b)
                fire_gathers(b, c)

        def pair(p, carry):
            step(2 * p, 0)
            step(2 * p + 1, 1)
            return carry

        lax.fori_loop(0, (nchunks + 2) // 2, pair, 0)

        wait_out(nchunks % 2)
        wait_out(1 - nchunks % 2)

    return emb_kernel


def kernel(input_ids, weight):
    batch, hist = input_ids.shape
    ids_flat = input_ids.reshape(-1).astype(jnp.int32)
    out = _build(batch * hist)(ids_flat, weight)
    return out.reshape(batch, hist, EMBEDDING_DIM)


# speedup vs baseline: 1.0462x; 1.0051x over previous
import functools

import jax
import jax.numpy as jnp
from jax import lax
from jax.experimental import pallas as pl
from jax.experimental.pallas import tpu as pltpu
from jax.experimental.pallas import tpu_sc as plsc

EMBEDDING_DIM = 64
IDX_PER_GATHER = 128
GATHERS_PER_CHUNK = 4
CHUNK = IDX_PER_GATHER * GATHERS_PER_CHUNK


def _build(num_ids: int):
    info = plsc.get_sparse_core_info()
    nc, ns = info.num_cores, info.num_subcores
    nw = nc * ns
    per_w = num_ids // nw
    assert per_w * nw == num_ids and per_w % CHUNK == 0
    nchunks = per_w // CHUNK
    mesh = plsc.VectorSubcoreMesh(core_axis_name="c", subcore_axis_name="s")

    @functools.partial(
        pl.kernel,
        mesh=mesh,
        out_type=jax.ShapeDtypeStruct((num_ids, EMBEDDING_DIM), jnp.float32),
        scratch_types=[
            pltpu.VMEM((2, CHUNK), jnp.int32),
            pltpu.VMEM((2, CHUNK, EMBEDDING_DIM), jnp.float32),
            pltpu.SemaphoreType.DMA((2,)),
            pltpu.SemaphoreType.DMA((2,)),
            pltpu.SemaphoreType.DMA((2,)),
        ],
        compiler_params=pltpu.CompilerParams(use_tc_tiling_on_sc=False),
    )
    def emb_kernel(ids_hbm, w_hbm, out_hbm, idx_v, rows_v, sem_i, sem_g, sem_o):
        wid = lax.axis_index("s") * nc + lax.axis_index("c")
        base = wid * per_w
        G = IDX_PER_GATHER

        def wait_idx(b):
            pltpu.make_async_copy(
                ids_hbm.at[pl.ds(0, CHUNK)], idx_v.at[b], sem_i.at[b]
            ).wait()

        def wait_gathers(b):
            for j in range(GATHERS_PER_CHUNK):
                pltpu.make_async_copy(
                    out_hbm.at[pl.ds(0, G)],
                    rows_v.at[b, pl.ds(j * G, G)],
                    sem_g.at[b],
                ).wait()

        def wait_out(b):
            pltpu.make_async_copy(
                rows_v.at[b], out_hbm.at[pl.ds(0, CHUNK)], sem_o.at[b]
            ).wait()

        def fire_gathers(b, c):
            for j in range(GATHERS_PER_CHUNK):
                pltpu.async_copy(
                    w_hbm.at[idx_v.at[b, pl.ds(j * G, G)]],
                    rows_v.at[b, pl.ds(j * G, G)],
                    sem_g.at[b],
                )

        pltpu.async_copy(ids_hbm.at[pl.ds(base, CHUNK)], idx_v.at[0], sem_i.at[0])
        pltpu.async_copy(
            ids_hbm.at[pl.ds(base + CHUNK, CHUNK)], idx_v.at[1], sem_i.at[1]
        )

        def step(c, b):
            @pl.when(c < nchunks)
            def _():
                @pl.when(c >= 2)
                def _():
                    wait_out(b)

                wait_idx(b)
                fire_gathers(b, c)

            @pl.when(jnp.logical_and(c >= 1, c <= nchunks))
            def _():
                wait_gathers(1 - b)
                pltpu.async_copy(
                    rows_v.at[1 - b],
                    out_hbm.at[pl.ds(base + (c - 1) * CHUNK, CHUNK)],
                    sem_o.at[1 - b],
                )

                @pl.when(c + 1 < nchunks)
                def _():
                    pltpu.async_copy(
                        ids_hbm.at[pl.ds(base + (c + 1) * CHUNK, CHUNK)],
                        idx_v.at[1 - b],
                        sem_i.at[1 - b],
                    )

        def pair(p, carry):
            step(2 * p, 0)
            step(2 * p + 1, 1)
            return carry

        lax.fori_loop(0, (nchunks + 2) // 2, pair, 0)

        wait_out(nchunks % 2)
        wait_out(1 - nchunks % 2)

    return emb_kernel


def kernel(input_ids, weight):
    batch, hist = input_ids.shape
    ids_flat = input_ids.reshape(-1).astype(jnp.int32)
    out = _build(batch * hist)(ids_flat, weight)
    return out.reshape(batch, hist, EMBEDDING_DIM)
